# baseline (speedup 1.0000x reference)
import functools

import jax
import jax.numpy as jnp
from jax import lax
from jax.experimental import pallas as pl
from jax.experimental.pallas import tpu as pltpu
from jax.experimental.pallas import tpu_sc as plsc

B = 1024
S = 200
D = 128
SP = 208
L = 16
NW = 32
SEQ_PER_W = B // NW
N_GROUPS = SP // L


def _sc_body(x_hbm, seg_hbm, tok_hbm, ps_hbm, gam_hbm, bet_hbm, out_hbm,
             idx_v, cidx_v, rows_v, rows2_v, gam_v, bet_v, sem):
    wid = lax.axis_index("s") * 2 + lax.axis_index("c")

    pltpu.sync_copy(gam_hbm, gam_v)
    pltpu.sync_copy(bet_hbm, bet_v)

    def seq_body(s, carry):
        b = wid * SEQ_PER_W + s
        pltpu.sync_copy(x_hbm.at[pl.ds(b * S, S)], idx_v.at[pl.ds(0, S)])
        pltpu.sync_copy(seg_hbm.at[pl.ds(b * S, S)], cidx_v.at[pl.ds(0, S)])

        for g in range(N_GROUPS):
            base = g * L
            sv = cidx_v[pl.ds(base, L)]
            pv = base + lax.iota(jnp.int32, L)
            cidx_v[pl.ds(base, L)] = 2 * pv + sv

        cps = [
            pltpu.async_copy(tok_hbm.at[idx_v.at[pl.ds(0, 104)]],
                             rows_v.at[pl.ds(0, 104)], sem),
            pltpu.async_copy(tok_hbm.at[idx_v.at[pl.ds(104, 96)]],
                             rows_v.at[pl.ds(104, 96)], sem),
            pltpu.async_copy(ps_hbm.at[cidx_v.at[pl.ds(0, 104)]],
                             rows2_v.at[pl.ds(0, 104)], sem),
            pltpu.async_copy(ps_hbm.at[cidx_v.at[pl.ds(104, 96)]],
                             rows2_v.at[pl.ds(104, 96)], sem),
        ]
        for cp in cps:
            cp.wait()

        zero = jnp.zeros((L,), jnp.float32)
        for g in range(N_GROUPS):
            tvec = g * L + lax.iota(jnp.int32, L)

            def p1(j, c):
                s1, s2 = c
                jv = jnp.zeros((L,), jnp.int32) + j
                v = (plsc.load_gather(rows_v, [tvec, jv])
                     + plsc.load_gather(rows2_v, [tvec, jv]))
                plsc.store_scatter(rows_v, [tvec, jv], v)
                return (s1 + v, s2 + v * v)

            s1, s2 = lax.fori_loop(0, D, p1, (zero, zero))
            mean = s1 * (1.0 / D)
            var = s2 * (1.0 / D) - mean * mean
            xv = var + 1e-5
            y = plsc.bitcast(jnp.int32(0x5F3759DF) - (plsc.bitcast(xv, jnp.int32) >> 1),
                             jnp.float32)
            for _ in range(3):
                y = y * (1.5 - 0.5 * xv * y * y)

            def p2(j, c):
                jv = jnp.zeros((L,), jnp.int32) + j
                v = plsc.load_gather(rows_v, [tvec, jv])
                gj = plsc.load_gather(gam_v, [jv])
                bj = plsc.load_gather(bet_v, [jv])
                o = (v - mean) * y * gj + bj
                plsc.store_scatter(rows2_v, [tvec, jv], o)
                return c

            lax.fori_loop(0, D, p2, 0)

        pltpu.sync_copy(rows2_v.at[pl.ds(0, S)], out_hbm.at[pl.ds(b * S, S)])
        return carry

    lax.fori_loop(0, SEQ_PER_W, seq_body, 0)


@jax.jit
def _embed_ln(x, seg, tok_embed, posseg, ln_gamma, ln_beta):
    mesh = plsc.VectorSubcoreMesh(core_axis_name="c", subcore_axis_name="s")
    f = pl.kernel(
        _sc_body,
        out_type=jax.ShapeDtypeStruct((B * S, D), jnp.float32),
        mesh=mesh,
        scratch_types=[
            pltpu.VMEM((SP,), jnp.int32),
            pltpu.VMEM((SP,), jnp.int32),
            pltpu.VMEM((SP, D), jnp.float32),
            pltpu.VMEM((SP, D), jnp.float32),
            pltpu.VMEM((D,), jnp.float32),
            pltpu.VMEM((D,), jnp.float32),
            pltpu.SemaphoreType.DMA,
        ],
        compiler_params=pltpu.CompilerParams(needs_layout_passes=False),
    )
    out = f(x.reshape(-1), seg.reshape(-1), tok_embed, posseg,
            ln_gamma, ln_beta)
    return out.reshape(B, S, D)


def kernel(x, seg, tok_embed, pos_embed, seg_embed, ln_gamma, ln_beta):
    posseg = (pos_embed[:, None, :] + seg_embed[None, :, :]).reshape(2 * S, D)
    return _embed_ln(x, seg, tok_embed, posseg, ln_gamma, ln_beta)
